# baseline (speedup 1.0000x reference)
import functools

import jax
import jax.numpy as jnp
from jax import lax
from jax.experimental import pallas as pl
from jax.experimental.pallas import tpu as pltpu
from jax.experimental.pallas import tpu_sc as plsc


_F32 = jnp.float32


def _enc_body(a_ref, w1_ref, b1_ref, w2_ref, b2_ref, w3_ref, b3_ref, ct_ref,
              idx_ref, P, S3):
    out1 = jnp.dot(a_ref[0], w1_ref[...], preferred_element_type=_F32)
    out1 = jnp.maximum(out1 + b1_ref[...], 0.0)

    P[...] = jnp.zeros(P.shape, _F32)
    for a in range(2):
        for b in range(2):
            ph = out1[(a * 2 + b) * 3136:(a * 2 + b + 1) * 3136, :]
            cbase = ((1 - a) * 2 + (1 - b)) * 32
            P[a:a + 56, b:b + 56, cbase:cbase + 32] = ph.reshape(56, 56, 32)

    acc2 = jnp.zeros((3136, 64), _F32)
    for sy in range(2):
        for sx in range(2):
            xs = P[sy:sy + 56, sx:sx + 56, :].reshape(3136, 128)
            acc2 = acc2 + jnp.dot(xs, w2_ref[sy * 2 + sx],
                                  preferred_element_type=_F32)
    h2 = jnp.maximum(acc2 + b2_ref[...], 0.0)

    S3[...] = jnp.zeros(S3.shape, _F32)
    S3[1:57, 1:57, :] = h2.reshape(56, 56, 64)
    acc3 = jnp.zeros((3136, 64), _F32)
    for ky in range(3):
        for kx in range(3):
            xs = S3[ky:ky + 56, kx:kx + 56, :].reshape(3136, 64)
            acc3 = acc3 + jnp.dot(xs, w3_ref[ky * 3 + kx],
                                  preferred_element_type=_F32)
    z_e = acc3 + b3_ref[...]

    ct = ct_ref[...]
    cn = jnp.sum(ct * ct, axis=0)
    z2 = jnp.sum(z_e * z_e, axis=1, keepdims=True)
    d = (z2 - 2.0 * jnp.dot(z_e, ct, preferred_element_type=_F32)) + cn[None, :]
    m = jnp.min(d, axis=1, keepdims=True)
    ii = lax.broadcasted_iota(jnp.int32, (3136, 512), 1)
    idx_ref[0, 0] = jnp.min(jnp.where(d <= m, ii, 512), axis=1)


def _enc_call(a1, w1, b1, w2, b2, w3, b3, ct):
    N = a1.shape[0]
    return pl.pallas_call(
        _enc_body,
        grid=(N,),
        in_specs=[
            pl.BlockSpec((1, 12544, 16), lambda n: (n, 0, 0)),
            pl.BlockSpec((16, 32), lambda n: (0, 0)),
            pl.BlockSpec((1, 32), lambda n: (0, 0)),
            pl.BlockSpec((4, 128, 64), lambda n: (0, 0, 0)),
            pl.BlockSpec((1, 64), lambda n: (0, 0)),
            pl.BlockSpec((9, 64, 64), lambda n: (0, 0, 0)),
            pl.BlockSpec((1, 64), lambda n: (0, 0)),
            pl.BlockSpec((64, 512), lambda n: (0, 0)),
        ],
        out_specs=pl.BlockSpec((1, 1, 3136), lambda n: (n, 0, 0)),
        out_shape=jax.ShapeDtypeStruct((N, 1, 3136), jnp.int32),
        scratch_shapes=[
            pltpu.VMEM((57, 57, 128), _F32),
            pltpu.VMEM((58, 58, 64), _F32),
        ],
        compiler_params=pltpu.CompilerParams(
            dimension_semantics=("parallel",)),
    )(a1, w1, b1.reshape(1, 32), w2, b2.reshape(1, 64), w3,
      b3.reshape(1, 64), ct)


def _dec_body(zq_ref, wd1_ref, bd1_ref, wd2_ref, bd2_ref, wd3_ref, bd3_ref,
              o_ref, S1, S2, P2):
    S1[...] = jnp.zeros(S1.shape, _F32)
    S1[1:57, 1:57, :] = zq_ref[0].reshape(56, 56, 64)
    acc1 = jnp.zeros((3136, 64), _F32)
    for ky in range(3):
        for kx in range(3):
            xs = S1[ky:ky + 56, kx:kx + 56, :].reshape(3136, 64)
            acc1 = acc1 + jnp.dot(xs, wd1_ref[ky * 3 + kx],
                                  preferred_element_type=_F32)
    y1 = jnp.maximum(acc1 + bd1_ref[...], 0.0)

    S2[...] = jnp.zeros(S2.shape, _F32)
    S2[1:57, 1:57, :] = y1.reshape(56, 56, 64)
    acc2 = jnp.zeros((3136, 128), _F32)
    for ky in range(3):
        for kx in range(3):
            xs = S2[ky:ky + 56, kx:kx + 56, :].reshape(3136, 64)
            acc2 = acc2 + jnp.dot(xs, wd2_ref[ky * 3 + kx],
                                  preferred_element_type=_F32)
    y2p = jnp.maximum(acc2 + bd2_ref[...], 0.0)

    P2[...] = jnp.zeros(P2.shape, _F32)
    for r in range(2):
        for s in range(2):
            ph = y2p[:, (r * 2 + s) * 32:(r * 2 + s + 1) * 32]
            cbase = ((1 - r) * 2 + (1 - s)) * 32
            P2[r:r + 56, s:s + 56, cbase:cbase + 32] = ph.reshape(56, 56, 32)

    acc3 = jnp.zeros((3136, 16), _F32)
    for sy in range(2):
        for sx in range(2):
            xs = P2[sy:sy + 56, sx:sx + 56, :].reshape(3136, 128)
            acc3 = acc3 + jnp.dot(xs, wd3_ref[sy * 2 + sx],
                                  preferred_element_type=_F32)
    o_ref[0] = jax.nn.sigmoid(acc3 + bd3_ref[...])


def _dec_call(zq, wd1, bd1, wd2, bd2, wd3, bd3):
    N = zq.shape[0]
    return pl.pallas_call(
        _dec_body,
        grid=(N,),
        in_specs=[
            pl.BlockSpec((1, 3136, 64), lambda n: (n, 0, 0)),
            pl.BlockSpec((9, 64, 64), lambda n: (0, 0, 0)),
            pl.BlockSpec((1, 64), lambda n: (0, 0)),
            pl.BlockSpec((9, 64, 128), lambda n: (0, 0, 0)),
            pl.BlockSpec((1, 128), lambda n: (0, 0)),
            pl.BlockSpec((4, 128, 16), lambda n: (0, 0, 0)),
            pl.BlockSpec((1, 16), lambda n: (0, 0)),
        ],
        out_specs=pl.BlockSpec((1, 3136, 16), lambda n: (n, 0, 0)),
        out_shape=jax.ShapeDtypeStruct((N, 3136, 16), _F32),
        scratch_shapes=[
            pltpu.VMEM((58, 58, 64), _F32),
            pltpu.VMEM((58, 58, 64), _F32),
            pltpu.VMEM((57, 57, 128), _F32),
        ],
        compiler_params=pltpu.CompilerParams(
            dimension_semantics=("parallel",)),
    )(zq, wd1, bd1.reshape(1, 64), wd2, bd2.reshape(1, 128), wd3,
      bd3.reshape(1, 16))


def _sc_gather_t(table, idx, B, D, n_chunks=2):
    V = table.shape[0]
    info = plsc.get_sparse_core_info()
    NW = info.num_cores * info.num_subcores
    L = info.num_lanes
    b_per_w = B // NW
    b_chunk = b_per_w // n_chunks
    n_groups = b_chunk // L
    mesh = plsc.VectorSubcoreMesh(core_axis_name="c", subcore_axis_name="s")

    @functools.partial(
        pl.kernel, mesh=mesh,
        out_type=jax.ShapeDtypeStruct((B * D,), _F32),
        scratch_types=[
            pltpu.VMEM((V * D,), _F32),
            pltpu.VMEM((b_chunk,), jnp.int32),
            pltpu.VMEM((b_chunk * D,), _F32),
        ],
        compiler_params=pltpu.CompilerParams(needs_layout_passes=False),
    )
    def k(table_hbm, idx_hbm, out_hbm, table_v, idx_v, out_v):
        wid = lax.axis_index("s") * info.num_cores + lax.axis_index("c")
        pltpu.sync_copy(table_hbm, table_v)
        for ch in range(n_chunks):
            base = wid * b_per_w + ch * b_chunk
            pltpu.sync_copy(idx_hbm.at[pl.ds(base, b_chunk)], idx_v)

            def body(g, _):
                row_base = idx_v[pl.ds(g * L, L)] * D
                out_base = g * (L * D)
                for c in range(D):
                    vals = plsc.load_gather(table_v, [row_base + c])
                    plsc.store_scatter(
                        out_v, [lax.iota(jnp.int32, L) * D + (out_base + c)],
                        vals)
                return _

            lax.fori_loop(0, n_groups, body, 0)
            pltpu.sync_copy(out_v, out_hbm.at[pl.ds(base * D, b_chunk * D)])

    return k(table.reshape(V * D), idx)


def _phase_conv_weights(w, CO):
    E = jnp.zeros((3, 2, 4), _F32)
    E = E.at[0, 0, 3].set(1.0).at[1, 0, 1].set(1.0)
    E = E.at[1, 1, 2].set(1.0).at[2, 1, 0].set(1.0)
    wp = jnp.einsum("kry,lsx,icyx->klirsc", E, E, w)
    CI = w.shape[0]
    return wp.reshape(9, CI, 4 * CO)


def kernel(x, enc_w1, enc_b1, enc_w2, enc_b2, enc_w3, enc_b3, codebook,
           dec_w1, dec_b1, dec_w2, dec_b2, dec_w3, dec_b3):
    N = x.shape[0]

    x0 = jnp.pad(x[:, 0, :, :][..., None], ((0, 0), (1, 1), (1, 1), (0, 0)))
    s2d = (x0.reshape(N, 113, 2, 113, 2, 1).transpose(0, 1, 3, 2, 4, 5)
           .reshape(N, 113, 113, 4))
    a1 = jnp.concatenate(
        [s2d[:, sy:sy + 112, sx:sx + 112, :]
         for sy in (0, 1) for sx in (0, 1)], axis=-1)
    a1 = (a1.reshape(N, 56, 2, 56, 2, 16).transpose(0, 2, 4, 1, 3, 5)
          .reshape(N, 12544, 16))

    w1 = enc_w1[:, 0].transpose(1, 2, 0)
    w1 = w1.reshape(2, 2, 2, 2, 32).transpose(0, 2, 1, 3, 4).reshape(16, 32)
    w2 = (enc_w2.transpose(2, 3, 1, 0).reshape(2, 2, 2, 2, 32, 64)
          .transpose(0, 2, 1, 3, 4, 5).reshape(4, 128, 64))
    w3 = enc_w3.transpose(2, 3, 1, 0).reshape(9, 64, 64)
    wd1 = jnp.flip(dec_w1, (2, 3)).transpose(1, 0, 2, 3)
    wd1 = wd1.transpose(2, 3, 1, 0).reshape(9, 64, 64)
    wd2 = _phase_conv_weights(dec_w2, 32)
    bd2 = jnp.tile(dec_b2, 4)
    wd3p = _phase_conv_weights(dec_w3, 1)
    wd3 = jnp.zeros((4, 128, 16), _F32)
    for sy in range(2):
        for sx in range(2):
            for al in range(2):
                for be in range(2):
                    for u in range(2):
                        for v in range(2):
                            ky = 2 * sy + al - u
                            kx = 2 * sx + be - v
                            if 0 <= ky <= 2 and 0 <= kx <= 2:
                                wd3 = wd3.at[
                                    sy * 2 + sx,
                                    al * 64 + be * 32:al * 64 + be * 32 + 32,
                                    (u * 2 + v) * 4:(u * 2 + v) * 4 + 4,
                                ].set(wd3p[ky * 3 + kx])
    bd3 = jnp.tile(dec_b3, 16)
    ct = codebook.T

    idx3 = _enc_call(a1, w1, enc_b1, w2, enc_b2, w3, enc_b3, ct)
    idx_flat = idx3.reshape(N * 3136)

    zq_flat = _sc_gather_t(codebook, idx_flat, N * 3136, 64)
    z_q_nhwc = zq_flat.reshape(N, 56, 56, 64)
    z_q = z_q_nhwc.transpose(0, 3, 1, 2)

    y3p = _dec_call(zq_flat.reshape(N, 3136, 64), wd1, dec_b1, wd2, bd2,
                    wd3, bd3)
    y3 = (y3p.reshape(N, 56, 56, 2, 2, 2, 2)
          .transpose(0, 1, 3, 5, 2, 4, 6).reshape(N, 224, 224))
    x_recon = y3[:, None, :, :]

    return (x_recon, z_q, idx_flat)


# speedup vs baseline: 1.2187x; 1.2187x over previous
import functools

import jax
import jax.numpy as jnp
from jax import lax
from jax.experimental import pallas as pl
from jax.experimental.pallas import tpu as pltpu
from jax.experimental.pallas import tpu_sc as plsc


_F32 = jnp.float32


def _enc_body(a_ref, w1_ref, b1_ref, w2_ref, b2_ref, w3_ref, b3_ref, ct_ref,
              idx_ref, P, S3):
    acc1 = jnp.zeros((3136, 128), _F32)
    for dl in range(2):
        for ep in range(2):
            xs = a_ref[0, dl:dl + 56, ep:ep + 56, :].reshape(3136, 16)
            acc1 = acc1 + jnp.dot(xs, w1_ref[dl * 2 + ep],
                                  preferred_element_type=_F32)
    out1 = jnp.maximum(acc1 + b1_ref[...], 0.0)

    P[...] = jnp.zeros(P.shape, _F32)
    for a in range(2):
        for b in range(2):
            ph = out1[:, (a * 2 + b) * 32:(a * 2 + b + 1) * 32]
            cbase = ((1 - a) * 2 + (1 - b)) * 32
            P[a:a + 56, b:b + 56, cbase:cbase + 32] = ph.reshape(56, 56, 32)

    acc2 = jnp.zeros((3136, 64), _F32)
    for sy in range(2):
        for sx in range(2):
            xs = P[sy:sy + 56, sx:sx + 56, :].reshape(3136, 128)
            acc2 = acc2 + jnp.dot(xs, w2_ref[sy * 2 + sx],
                                  preferred_element_type=_F32)
    h2 = jnp.maximum(acc2 + b2_ref[...], 0.0)

    S3[...] = jnp.zeros(S3.shape, _F32)
    S3[1:57, 1:57, :] = h2.reshape(56, 56, 64)
    acc3 = jnp.zeros((3136, 64), _F32)
    for ky in range(3):
        for kx in range(3):
            xs = S3[ky:ky + 56, kx:kx + 56, :].reshape(3136, 64)
            acc3 = acc3 + jnp.dot(xs, w3_ref[ky * 3 + kx],
                                  preferred_element_type=_F32)
    z_e = acc3 + b3_ref[...]

    ct = ct_ref[...]
    cn = jnp.sum(ct * ct, axis=0)
    z2 = jnp.sum(z_e * z_e, axis=1, keepdims=True)
    d = (z2 - 2.0 * jnp.dot(z_e, ct, preferred_element_type=_F32)) + cn[None, :]
    m = jnp.min(d, axis=1, keepdims=True)
    ii = lax.broadcasted_iota(jnp.int32, (3136, 512), 1)
    idx_ref[0, 0] = jnp.min(jnp.where(d <= m, ii, 512), axis=1)


def _enc_call(a1, w1, b1, w2, b2, w3, b3, ct):
    N = a1.shape[0]
    return pl.pallas_call(
        _enc_body,
        grid=(N,),
        in_specs=[
            pl.BlockSpec((1, 57, 57, 16), lambda n: (n, 0, 0, 0)),
            pl.BlockSpec((4, 16, 128), lambda n: (0, 0, 0)),
            pl.BlockSpec((1, 128), lambda n: (0, 0)),
            pl.BlockSpec((4, 128, 64), lambda n: (0, 0, 0)),
            pl.BlockSpec((1, 64), lambda n: (0, 0)),
            pl.BlockSpec((9, 64, 64), lambda n: (0, 0, 0)),
            pl.BlockSpec((1, 64), lambda n: (0, 0)),
            pl.BlockSpec((64, 512), lambda n: (0, 0)),
        ],
        out_specs=pl.BlockSpec((1, 1, 3136), lambda n: (n, 0, 0)),
        out_shape=jax.ShapeDtypeStruct((N, 1, 3136), jnp.int32),
        scratch_shapes=[
            pltpu.VMEM((57, 57, 128), _F32),
            pltpu.VMEM((58, 58, 64), _F32),
        ],
        compiler_params=pltpu.CompilerParams(
            dimension_semantics=("parallel",)),
    )(a1, w1, jnp.tile(b1, 4).reshape(1, 128), w2, b2.reshape(1, 64), w3,
      b3.reshape(1, 64), ct)


def _dec_body(zq_ref, wd1_ref, bd1_ref, wd2_ref, bd2_ref, wd3_ref, bd3_ref,
              o_ref, S1, S2, P2):
    S1[...] = jnp.zeros(S1.shape, _F32)
    S1[1:57, 1:57, :] = zq_ref[0].reshape(56, 56, 64)
    acc1 = jnp.zeros((3136, 64), _F32)
    for ky in range(3):
        for kx in range(3):
            xs = S1[ky:ky + 56, kx:kx + 56, :].reshape(3136, 64)
            acc1 = acc1 + jnp.dot(xs, wd1_ref[ky * 3 + kx],
                                  preferred_element_type=_F32)
    y1 = jnp.maximum(acc1 + bd1_ref[...], 0.0)

    S2[...] = jnp.zeros(S2.shape, _F32)
    S2[1:57, 1:57, :] = y1.reshape(56, 56, 64)
    acc2 = jnp.zeros((3136, 128), _F32)
    for ky in range(3):
        for kx in range(3):
            xs = S2[ky:ky + 56, kx:kx + 56, :].reshape(3136, 64)
            acc2 = acc2 + jnp.dot(xs, wd2_ref[ky * 3 + kx],
                                  preferred_element_type=_F32)
    y2p = jnp.maximum(acc2 + bd2_ref[...], 0.0)

    P2[...] = jnp.zeros(P2.shape, _F32)
    for r in range(2):
        for s in range(2):
            ph = y2p[:, (r * 2 + s) * 32:(r * 2 + s + 1) * 32]
            cbase = ((1 - r) * 2 + (1 - s)) * 32
            P2[r:r + 56, s:s + 56, cbase:cbase + 32] = ph.reshape(56, 56, 32)

    acc3 = jnp.zeros((3136, 16), _F32)
    for sy in range(2):
        for sx in range(2):
            xs = P2[sy:sy + 56, sx:sx + 56, :].reshape(3136, 128)
            acc3 = acc3 + jnp.dot(xs, wd3_ref[sy * 2 + sx],
                                  preferred_element_type=_F32)
    o_ref[0] = jax.nn.sigmoid(acc3 + bd3_ref[...])


def _dec_call(zq, wd1, bd1, wd2, bd2, wd3, bd3):
    N = zq.shape[0]
    return pl.pallas_call(
        _dec_body,
        grid=(N,),
        in_specs=[
            pl.BlockSpec((1, 3136, 64), lambda n: (n, 0, 0)),
            pl.BlockSpec((9, 64, 64), lambda n: (0, 0, 0)),
            pl.BlockSpec((1, 64), lambda n: (0, 0)),
            pl.BlockSpec((9, 64, 128), lambda n: (0, 0, 0)),
            pl.BlockSpec((1, 128), lambda n: (0, 0)),
            pl.BlockSpec((4, 128, 16), lambda n: (0, 0, 0)),
            pl.BlockSpec((1, 16), lambda n: (0, 0)),
        ],
        out_specs=pl.BlockSpec((1, 3136, 16), lambda n: (n, 0, 0)),
        out_shape=jax.ShapeDtypeStruct((N, 3136, 16), _F32),
        scratch_shapes=[
            pltpu.VMEM((58, 58, 64), _F32),
            pltpu.VMEM((58, 58, 64), _F32),
            pltpu.VMEM((57, 57, 128), _F32),
        ],
        compiler_params=pltpu.CompilerParams(
            dimension_semantics=("parallel",)),
    )(zq, wd1, bd1.reshape(1, 64), wd2, bd2.reshape(1, 128), wd3,
      bd3.reshape(1, 16))


def _sc_gather_t(table, idx, B, D, n_chunks=2):
    V = table.shape[0]
    info = plsc.get_sparse_core_info()
    NW = info.num_cores * info.num_subcores
    L = info.num_lanes
    b_per_w = B // NW
    b_chunk = b_per_w // n_chunks
    n_groups = b_chunk // L
    mesh = plsc.VectorSubcoreMesh(core_axis_name="c", subcore_axis_name="s")

    @functools.partial(
        pl.kernel, mesh=mesh,
        out_type=jax.ShapeDtypeStruct((B * D,), _F32),
        scratch_types=[
            pltpu.VMEM((V * D,), _F32),
            pltpu.VMEM((b_chunk,), jnp.int32),
            pltpu.VMEM((b_chunk * D,), _F32),
        ],
        compiler_params=pltpu.CompilerParams(needs_layout_passes=False),
    )
    def k(table_hbm, idx_hbm, out_hbm, table_v, idx_v, out_v):
        wid = lax.axis_index("s") * info.num_cores + lax.axis_index("c")
        pltpu.sync_copy(table_hbm, table_v)
        iv = lax.iota(jnp.int32, L) * D
        for ch in range(n_chunks):
            base = wid * b_per_w + ch * b_chunk
            pltpu.sync_copy(idx_hbm.at[pl.ds(base, b_chunk)], idx_v)

            @plsc.parallel_loop(0, n_groups, 1, unroll=4)
            def body(g):
                row_base = idx_v[pl.ds(g * L, L)] * D
                ov = iv + g * (L * D)
                for c in range(D):
                    vals = plsc.load_gather(table_v, [row_base + c])
                    plsc.store_scatter(out_v, [ov + c], vals)

            pltpu.sync_copy(out_v, out_hbm.at[pl.ds(base * D, b_chunk * D)])

    return k(table.reshape(V * D), idx)


def _phase_conv_weights(w, CO):
    E = jnp.zeros((3, 2, 4), _F32)
    E = E.at[0, 0, 3].set(1.0).at[1, 0, 1].set(1.0)
    E = E.at[1, 1, 2].set(1.0).at[2, 1, 0].set(1.0)
    wp = jnp.einsum("kry,lsx,icyx->klirsc", E, E, w)
    CI = w.shape[0]
    return wp.reshape(9, CI, 4 * CO)


def kernel(x, enc_w1, enc_b1, enc_w2, enc_b2, enc_w3, enc_b3, codebook,
           dec_w1, dec_b1, dec_w2, dec_b2, dec_w3, dec_b3):
    N = x.shape[0]

    xp = jnp.pad(x[:, 0, :, :], ((0, 0), (1, 3), (1, 3)))
    a1 = (xp.reshape(N, 57, 4, 57, 4).transpose(0, 1, 3, 2, 4)
          .reshape(N, 57, 57, 16))

    w1 = jnp.zeros((2, 2, 4, 4, 2, 2, 32), _F32)
    for dl in range(2):
        for j in range(4):
            for a in range(2):
                ky = 4 * dl + j - 2 * a
                if not 0 <= ky <= 3:
                    continue
                for ep in range(2):
                    for i in range(4):
                        for b in range(2):
                            kx = 4 * ep + i - 2 * b
                            if 0 <= kx <= 3:
                                w1 = w1.at[dl, ep, j, i, a, b, :].set(
                                    enc_w1[:, 0, ky, kx])
    w1 = w1.reshape(4, 16, 128)
    w2 = (enc_w2.transpose(2, 3, 1, 0).reshape(2, 2, 2, 2, 32, 64)
          .transpose(0, 2, 1, 3, 4, 5).reshape(4, 128, 64))
    w3 = enc_w3.transpose(2, 3, 1, 0).reshape(9, 64, 64)
    wd1 = jnp.flip(dec_w1, (2, 3)).transpose(1, 0, 2, 3)
    wd1 = wd1.transpose(2, 3, 1, 0).reshape(9, 64, 64)
    wd2 = _phase_conv_weights(dec_w2, 32)
    bd2 = jnp.tile(dec_b2, 4)
    wd3p = _phase_conv_weights(dec_w3, 1)
    wd3 = jnp.zeros((4, 128, 16), _F32)
    for sy in range(2):
        for sx in range(2):
            for al in range(2):
                for be in range(2):
                    for u in range(2):
                        for v in range(2):
                            ky = 2 * sy + al - u
                            kx = 2 * sx + be - v
                            if 0 <= ky <= 2 and 0 <= kx <= 2:
                                wd3 = wd3.at[
                                    sy * 2 + sx,
                                    al * 64 + be * 32:al * 64 + be * 32 + 32,
                                    (u * 2 + v) * 4:(u * 2 + v) * 4 + 4,
                                ].set(wd3p[ky * 3 + kx])
    bd3 = jnp.tile(dec_b3, 16)
    ct = codebook.T

    idx3 = _enc_call(a1, w1, enc_b1, w2, enc_b2, w3, enc_b3, ct)
    idx_flat = idx3.reshape(N * 3136)

    zq_flat = _sc_gather_t(codebook, idx_flat, N * 3136, 64)
    z_q_nhwc = zq_flat.reshape(N, 56, 56, 64)
    z_q = z_q_nhwc.transpose(0, 3, 1, 2)

    y3p = _dec_call(zq_flat.reshape(N, 3136, 64), wd1, dec_b1, wd2, bd2,
                    wd3, bd3)
    y3 = (y3p.reshape(N, 56, 56, 2, 2, 2, 2)
          .transpose(0, 1, 3, 5, 2, 4, 6).reshape(N, 224, 224))
    x_recon = y3[:, None, :, :]

    return (x_recon, z_q, idx_flat)
